# baseline (speedup 1.0000x reference)
import functools

import jax
import jax.numpy as jnp
from jax import lax
from jax.experimental import pallas as pl
from jax.experimental.pallas import tpu as pltpu
from jax.experimental.pallas import tpu_sc as plsc

B = 4
S = 2048
D = 768
L = 16
NC = 2
NS = 16
NW = NC * NS
POS_W = S // NW
CP = 8
CHUNKS = POS_W // CP
ROWS = B * CP
DV = D // L
NSLOT = 3


def _emb_lookup_body(
    x_hbm, tab_hbm, pos_hbm, out_hbm, idx_raw, idx_v, rows_v, pos_v, sem_pos, *sems
):
    sg = sems[0:NSLOT]
    so = sems[NSLOT:2 * NSLOT]
    wid = lax.axis_index("s") * NC + lax.axis_index("c")
    p_w = wid * POS_W
    pos_cp = pltpu.async_copy(pos_hbm.at[pl.ds(p_w, POS_W)], pos_v, sem_pos)
    for b in range(B):
        pltpu.sync_copy(x_hbm.at[b, pl.ds(p_w, POS_W)], idx_raw.at[b])
    lane = jax.lax.iota(jnp.int32, 16)
    offv = lane + jnp.where(lane >= CP, (B - 1) * CP, 0)
    for b in range(B):
        for h in range(POS_W // 16):
            v = idx_raw[b, pl.ds(16 * h, 16)]
            base = (2 * h) * ROWS + b * CP
            plsc.store_scatter(idx_v, [offv + base], v)

    def start_in(c):
        slot = c % NSLOT
        return pltpu.async_copy(
            tab_hbm.at[idx_v.at[pl.ds(c * ROWS, ROWS)]], rows_v.at[slot], sg[slot]
        )

    def start_out(c):
        slot = c % NSLOT
        return [
            pltpu.async_copy(
                rows_v.at[slot, pl.ds(b * CP, CP)],
                out_hbm.at[b, pl.ds(p_w + c * CP, CP)],
                so[slot],
            )
            for b in range(B)
        ]

    in_d = {0: start_in(0), 1: start_in(1)}
    out_d = {}
    in_d[0].wait()
    pos_cp.wait()
    for c in range(CHUNKS):
        slot = c % NSLOT
        nxt = c + 2
        if nxt < CHUNKS:
            if nxt - NSLOT >= 0:
                for dsc in out_d[nxt - NSLOT]:
                    dsc.wait()
            in_d[nxt] = start_in(nxt)
        if c > 0:
            with jax.named_scope("wait_in"):
                in_d[c].wait()

        def d_body(d, carry):
            col = d * L
            for i in range(CP):
                pv = pos_v[c * CP + i, pl.ds(col, L)]
                for b in range(B):
                    plsc.addupdate(
                        rows_v.at[slot, b * CP + i, pl.ds(col, L)], pv
                    )
            return carry

        with jax.named_scope("add"):
            lax.fori_loop(0, DV, d_body, 0)
        out_d[c] = start_out(c)

    with jax.named_scope("drain"):
        for c in range(max(0, CHUNKS - NSLOT), CHUNKS):
            for dsc in out_d[c]:
                dsc.wait()


@functools.cache
def _build(interpret: bool = False):
    mesh = plsc.VectorSubcoreMesh(
        core_axis_name="c", subcore_axis_name="s", num_cores=NC, num_subcores=NS
    )
    return pl.kernel(
        _emb_lookup_body,
        out_type=jax.ShapeDtypeStruct((B, S, D), jnp.float32),
        mesh=mesh,
        scratch_types=[
            pltpu.VMEM((B, POS_W), jnp.int32),
            pltpu.VMEM((CHUNKS * ROWS,), jnp.int32),
            pltpu.VMEM((NSLOT, ROWS, D), jnp.float32),
            pltpu.VMEM((POS_W, D), jnp.float32),
            pltpu.SemaphoreType.DMA,
        ] + [pltpu.SemaphoreType.DMA] * (2 * NSLOT),
        compiler_params=pltpu.CompilerParams(needs_layout_passes=False),
        interpret=interpret,
    )


def kernel(x, input_embeddings, positional_embeddings):
    return _build()(x.astype(jnp.int32), input_embeddings, positional_embeddings)


# speedup vs baseline: 1.0158x; 1.0158x over previous
import functools

import jax
import jax.numpy as jnp
from jax import lax
from jax.experimental import pallas as pl
from jax.experimental.pallas import tpu as pltpu
from jax.experimental.pallas import tpu_sc as plsc

B = 4
S = 2048
D = 768
L = 16
NC = 2
NS = 16
NW = NC * NS
POS_W = S // NW
CP = 8
CHUNKS = POS_W // CP
ROWS = B * CP
DV = D // L
NSLOT = 3


def _emb_lookup_body(
    xt_hbm, tab_hbm, pos_hbm, out_hbm, idx_v, rows_v, pos_v, sem_pos, *sems
):
    sg = sems[0:NSLOT]
    so = sems[NSLOT:2 * NSLOT]
    wid = lax.axis_index("s") * NC + lax.axis_index("c")
    p_w = wid * POS_W
    pos_cp = pltpu.async_copy(pos_hbm.at[pl.ds(p_w, POS_W)], pos_v, sem_pos)
    pltpu.sync_copy(xt_hbm.at[wid], idx_v)

    def start_in(c):
        slot = c % NSLOT
        return pltpu.async_copy(
            tab_hbm.at[idx_v.at[c]], rows_v.at[slot], sg[slot]
        )

    def start_out(c):
        slot = c % NSLOT
        return [
            pltpu.async_copy(
                rows_v.at[slot, pl.ds(b * CP, CP)],
                out_hbm.at[pl.ds(b * S + p_w + c * CP, CP)],
                so[slot],
            )
            for b in range(B)
        ]

    in_d = {0: start_in(0), 1: start_in(1)}
    out_d = {}
    in_d[0].wait()
    pos_cp.wait()
    for c in range(CHUNKS):
        slot = c % NSLOT
        nxt = c + 2
        if nxt < CHUNKS:
            if nxt - NSLOT >= 0:
                for dsc in out_d[nxt - NSLOT]:
                    dsc.wait()
            in_d[nxt] = start_in(nxt)
        if c > 0:
            with jax.named_scope("wait_in"):
                in_d[c].wait()

        def d_body(d, carry):
            col = d * L
            for i in range(CP):
                pv = pos_v[c * CP + i, pl.ds(col, L)]
                for b in range(B):
                    plsc.addupdate(
                        rows_v.at[slot, b * CP + i, pl.ds(col, L)], pv
                    )
            return carry

        with jax.named_scope("add"):
            lax.fori_loop(0, DV, d_body, 0)
        out_d[c] = start_out(c)

    with jax.named_scope("drain"):
        for c in range(max(0, CHUNKS - NSLOT), CHUNKS):
            for dsc in out_d[c]:
                dsc.wait()


@functools.cache
def _build(interpret: bool = False):
    mesh = plsc.VectorSubcoreMesh(
        core_axis_name="c", subcore_axis_name="s", num_cores=NC, num_subcores=NS
    )
    return pl.kernel(
        _emb_lookup_body,
        out_type=jax.ShapeDtypeStruct((B * S, D), jnp.float32),
        mesh=mesh,
        scratch_types=[
            pltpu.VMEM((CHUNKS, ROWS), jnp.int32),
            pltpu.VMEM((NSLOT, ROWS, D), jnp.float32),
            pltpu.VMEM((POS_W, D), jnp.float32),
            pltpu.SemaphoreType.DMA,
        ] + [pltpu.SemaphoreType.DMA] * (2 * NSLOT),
        interpret=interpret,
    )


def kernel(x, input_embeddings, positional_embeddings):
    xt = (
        x.astype(jnp.int32)
        .reshape(B, NW, CHUNKS, CP)
        .transpose(1, 2, 0, 3)
        .reshape(NW, CHUNKS, ROWS)
    )
    out = _build()(xt, input_embeddings, positional_embeddings)
    return out.reshape(B, S, D)
